# baseline (speedup 1.0000x reference)
import functools

import jax
import jax.numpy as jnp
from jax import lax
from jax.experimental import pallas as pl
from jax.experimental.pallas import tpu as pltpu
from jax.experimental.pallas import tpu_sc as plsc

N = 10000
D = 128
E = 320000
NC = 2
NS = 16
NW = NC * NS
CH = 96
NCH = 106
EPW = NCH * CH
PADE = NW * EPW
ACC_ROWS = 10112
ROWS_PT = ACC_ROWS // NS
OUT_PT = 624
OUT_REM = N - NS * OUT_PT


def _sc_agg_body(x_hbm, srcp_hbm, dstp_hbm, out_hbm,
                 sv0, sv1, dv0, dv1, r0, r1, acc, si0, si1, sg0, sg1):
    sv = (sv0, sv1)
    dv = (dv0, dv1)
    rows = (r0, r1)
    si = (si0, si1)
    sg = (sg0, sg1)
    cid = lax.axis_index("c")
    sid = lax.axis_index("s")
    base = (cid * NS + sid) * EPW
    zero16 = jnp.zeros((16,), jnp.float32)

    def idx_load(j, b):
        off = base + j * CH
        pltpu.async_copy(srcp_hbm.at[pl.ds(off, CH)], sv[b], si[b])
        pltpu.async_copy(dstp_hbm.at[pl.ds(off, CH)], dv[b], si[b])

    def idx_wait(j, b):
        off = base + j * CH
        pltpu.make_async_copy(srcp_hbm.at[pl.ds(off, CH)], sv[b], si[b]).wait()
        pltpu.make_async_copy(dstp_hbm.at[pl.ds(off, CH)], dv[b], si[b]).wait()

    def gather_start(b):
        pltpu.async_copy(x_hbm.at[sv[b]], rows[b], sg[b])

    def gather_wait(b):
        pltpu.make_async_copy(x_hbm.at[sv[b]], rows[b], sg[b]).wait()

    def scatter(b):
        pltpu.sync_copy(rows[b], acc.at[dv[b]], add=True)

    def zfill(i, _):
        r0[i // (D // 16), pl.ds((i % (D // 16)) * 16, 16)] = zero16
        return 0
    lax.fori_loop(0, CH * (D // 16), zfill, 0)
    base_r = sid * ROWS_PT
    for t in range(ROWS_PT // CH):
        pltpu.sync_copy(r0, acc.at[pl.ds(base_r + t * CH, CH)])
    zrem = ROWS_PT - (ROWS_PT // CH) * CH
    if zrem:
        pltpu.sync_copy(r0.at[pl.ds(0, zrem)],
                        acc.at[pl.ds(base_r + (ROWS_PT // CH) * CH, zrem)])

    plsc.subcore_barrier()

    idx_load(0, 0)
    idx_load(1, 1)
    idx_wait(0, 0)
    gather_start(0)

    def pair(t, _):
        for b in (0, 1):
            j = 2 * t + b
            nb = 1 - b
            idx_wait(j + 1, nb)
            gather_start(nb)
            gather_wait(b)
            scatter(b)
            idx_load(j + 2, b)
        return 0
    lax.fori_loop(0, (NCH - 2) // 2, pair, 0)

    idx_wait(NCH - 1, 1)
    gather_start(1)
    gather_wait(0)
    scatter(0)
    gather_wait(1)
    scatter(1)

    plsc.subcore_barrier()

    pltpu.sync_copy(acc.at[pl.ds(sid * OUT_PT, OUT_PT)],
                    out_hbm.at[cid].at[pl.ds(sid * OUT_PT, OUT_PT)])

    @pl.when(sid == 0)
    def _tail():
        pltpu.sync_copy(acc.at[pl.ds(NS * OUT_PT, OUT_REM)],
                        out_hbm.at[cid].at[pl.ds(NS * OUT_PT, OUT_REM)])


_sc_agg = pl.kernel(
    _sc_agg_body,
    out_type=jax.ShapeDtypeStruct((NC, N, D), jnp.float32),
    mesh=plsc.VectorSubcoreMesh(core_axis_name="c", subcore_axis_name="s",
                                num_cores=NC, num_subcores=NS),
    scratch_types=[
        pltpu.VMEM((CH,), jnp.int32),
        pltpu.VMEM((CH,), jnp.int32),
        pltpu.VMEM((CH,), jnp.int32),
        pltpu.VMEM((CH,), jnp.int32),
        pltpu.VMEM((CH, D), jnp.float32),
        pltpu.VMEM((CH, D), jnp.float32),
        pltpu.VMEM_SHARED((ACC_ROWS, D), jnp.float32),
        pltpu.SemaphoreType.DMA,
        pltpu.SemaphoreType.DMA,
        pltpu.SemaphoreType.DMA,
        pltpu.SemaphoreType.DMA,
    ],
    name="sc_agg",
)


def _tc_combine_body(relu, p0, p1, d0, d1, xr, wl, wr, bb, o):
    deg = d0[:, 0:1] + d1[:, 0:1]
    inv = 1.0 / jnp.maximum(deg, 1.0)
    mean = (p0[...] + p1[...]) * inv
    acc = jnp.dot(mean, wl[...], preferred_element_type=jnp.float32)
    acc = acc + jnp.dot(xr[...], wr[...], preferred_element_type=jnp.float32)
    acc = acc + bb[...]
    o[...] = jnp.maximum(acc, 0.0) if relu else acc


_TC_R = 1000


def _make_tc_combine(relu):
    row = lambda i: (i, 0)
    fixed = lambda i: (0, 0)
    return pl.pallas_call(
        functools.partial(_tc_combine_body, relu),
        grid=(N // _TC_R,),
        in_specs=[
            pl.BlockSpec((_TC_R, D), row),
            pl.BlockSpec((_TC_R, D), row),
            pl.BlockSpec((_TC_R, 16), row),
            pl.BlockSpec((_TC_R, 16), row),
            pl.BlockSpec((_TC_R, D), row),
            pl.BlockSpec((D, D), fixed),
            pl.BlockSpec((D, D), fixed),
            pl.BlockSpec((1, D), fixed),
        ],
        out_specs=pl.BlockSpec((_TC_R, D), row),
        out_shape=jax.ShapeDtypeStruct((N, D), jnp.float32),
    )


_tc_relu = _make_tc_combine(True)
_tc_plain = _make_tc_combine(False)


def kernel(x, edge_index, W_l1, W_r1, b1, W_l2, W_r2, b2, W_l3, W_r3, b3):
    src = edge_index[0]
    dst = edge_index[1]
    srcp = jnp.concatenate([src, jnp.zeros((PADE - E,), jnp.int32)])
    dstp = jnp.concatenate([dst, jnp.full((PADE - E,), N, jnp.int32)])
    ones = jnp.ones((N, D), jnp.float32)
    deg = _sc_agg(ones, srcp, dstp)
    d0, d1 = deg[0][:, :16], deg[1][:, :16]
    part1 = _sc_agg(x, srcp, dstp)
    h1 = _tc_relu(part1[0], part1[1], d0, d1, x,
                  W_l1.T, W_r1.T, b1.reshape(1, D))
    part2 = _sc_agg(h1, srcp, dstp)
    h2 = _tc_relu(part2[0], part2[1], d0, d1, h1,
                  W_l2.T, W_r2.T, b2.reshape(1, D))
    part3 = _sc_agg(h2, srcp, dstp)
    h3 = _tc_plain(part3[0], part3[1], d0, d1, h2,
                   W_l3.T, W_r3.T, b3.reshape(1, D))
    return h3


# speedup vs baseline: 1.1870x; 1.1870x over previous
import functools

import jax
import jax.numpy as jnp
from jax import lax
from jax.experimental import pallas as pl
from jax.experimental.pallas import tpu as pltpu
from jax.experimental.pallas import tpu_sc as plsc

N = 10000
D = 128
E = 320000
NC = 2
NS = 16
NW = NC * NS
CH = 96
NCH = 106
EPW = NCH * CH
PADE = NW * EPW
ACC_ROWS = 10112
ROWS_PT = ACC_ROWS // NS
OUT_PT = 624
OUT_REM = N - NS * OUT_PT


def _sc_agg_body(do_gather, x_hbm, srcp_hbm, dstp_hbm, out_hbm,
                 sv0, sv1, dv0, dv1, r0, r1, acc, si0, si1, sg0, sg1):
    sv = (sv0, sv1)
    dv = (dv0, dv1)
    rows = (r0, r1)
    si = (si0, si1)
    sg = (sg0, sg1)
    cid = lax.axis_index("c")
    sid = lax.axis_index("s")
    base = (cid * NS + sid) * EPW
    zero16 = jnp.zeros((16,), jnp.float32)

    def idx_load(j, b):
        off = base + j * CH
        pltpu.async_copy(srcp_hbm.at[pl.ds(off, CH)], sv[b], si[b])
        pltpu.async_copy(dstp_hbm.at[pl.ds(off, CH)], dv[b], si[b])

    def idx_wait(j, b):
        off = base + j * CH
        pltpu.make_async_copy(srcp_hbm.at[pl.ds(off, CH)], sv[b], si[b]).wait()
        pltpu.make_async_copy(dstp_hbm.at[pl.ds(off, CH)], dv[b], si[b]).wait()

    def gather_start(b):
        if do_gather:
            pltpu.async_copy(x_hbm.at[sv[b]], rows[b], sg[b])

    def gather_wait(b):
        if do_gather:
            pltpu.make_async_copy(x_hbm.at[sv[b]], rows[b], sg[b]).wait()

    def scatter(b):
        pltpu.sync_copy(rows[b], acc.at[dv[b]], add=True)

    def zfill(i, _):
        r0[i // (D // 16), pl.ds((i % (D // 16)) * 16, 16)] = zero16
        return 0
    lax.fori_loop(0, CH * (D // 16), zfill, 0)
    base_r = sid * ROWS_PT
    for t in range(ROWS_PT // CH):
        pltpu.sync_copy(r0, acc.at[pl.ds(base_r + t * CH, CH)])
    zrem = ROWS_PT - (ROWS_PT // CH) * CH
    if zrem:
        pltpu.sync_copy(r0.at[pl.ds(0, zrem)],
                        acc.at[pl.ds(base_r + (ROWS_PT // CH) * CH, zrem)])

    if not do_gather:
        one16 = jnp.ones((16,), jnp.float32)

        def ofill(i, _):
            r0[i // (D // 16), pl.ds((i % (D // 16)) * 16, 16)] = one16
            r1[i // (D // 16), pl.ds((i % (D // 16)) * 16, 16)] = one16
            return 0
        lax.fori_loop(0, CH * (D // 16), ofill, 0)

    plsc.subcore_barrier()

    idx_load(0, 0)
    idx_load(1, 1)
    idx_wait(0, 0)
    gather_start(0)

    def pair(t, _):
        for b in (0, 1):
            j = 2 * t + b
            nb = 1 - b
            idx_wait(j + 1, nb)
            gather_start(nb)
            gather_wait(b)
            scatter(b)
            idx_load(j + 2, b)
        return 0
    lax.fori_loop(0, (NCH - 2) // 2, pair, 0)

    idx_wait(NCH - 1, 1)
    gather_start(1)
    gather_wait(0)
    scatter(0)
    gather_wait(1)
    scatter(1)

    plsc.subcore_barrier()

    pltpu.sync_copy(acc.at[pl.ds(sid * OUT_PT, OUT_PT)],
                    out_hbm.at[cid].at[pl.ds(sid * OUT_PT, OUT_PT)])

    @pl.when(sid == 0)
    def _tail():
        pltpu.sync_copy(acc.at[pl.ds(NS * OUT_PT, OUT_REM)],
                        out_hbm.at[cid].at[pl.ds(NS * OUT_PT, OUT_REM)])


def _make_sc(do_gather):
    return pl.kernel(
        functools.partial(_sc_agg_body, do_gather),
        out_type=jax.ShapeDtypeStruct((NC, N, D), jnp.float32),
        mesh=plsc.VectorSubcoreMesh(core_axis_name="c", subcore_axis_name="s",
                                    num_cores=NC, num_subcores=NS),
        scratch_types=[
            pltpu.VMEM((CH,), jnp.int32),
            pltpu.VMEM((CH,), jnp.int32),
            pltpu.VMEM((CH,), jnp.int32),
            pltpu.VMEM((CH,), jnp.int32),
            pltpu.VMEM((CH, D), jnp.float32),
            pltpu.VMEM((CH, D), jnp.float32),
            pltpu.VMEM_SHARED((ACC_ROWS, D), jnp.float32),
            pltpu.SemaphoreType.DMA,
            pltpu.SemaphoreType.DMA,
            pltpu.SemaphoreType.DMA,
            pltpu.SemaphoreType.DMA,
        ],
        name="sc_agg" if do_gather else "sc_deg",
    )


_sc_agg = _make_sc(True)
_sc_deg = _make_sc(False)


def _tc_combine_body(relu, p0, p1, d0, d1, xr, wl, wr, bb, o):
    deg = d0[:, 0:1] + d1[:, 0:1]
    inv = 1.0 / jnp.maximum(deg, 1.0)
    mean = (p0[...] + p1[...]) * inv
    acc = jnp.dot(mean, wl[...], preferred_element_type=jnp.float32)
    acc = acc + jnp.dot(xr[...], wr[...], preferred_element_type=jnp.float32)
    acc = acc + bb[...]
    o[...] = jnp.maximum(acc, 0.0) if relu else acc


_TC_R = 1000


def _make_tc_combine(relu):
    row = lambda i: (i, 0)
    fixed = lambda i: (0, 0)
    return pl.pallas_call(
        functools.partial(_tc_combine_body, relu),
        grid=(N // _TC_R,),
        in_specs=[
            pl.BlockSpec((_TC_R, D), row),
            pl.BlockSpec((_TC_R, D), row),
            pl.BlockSpec((_TC_R, 16), row),
            pl.BlockSpec((_TC_R, 16), row),
            pl.BlockSpec((_TC_R, D), row),
            pl.BlockSpec((D, D), fixed),
            pl.BlockSpec((D, D), fixed),
            pl.BlockSpec((1, D), fixed),
        ],
        out_specs=pl.BlockSpec((_TC_R, D), row),
        out_shape=jax.ShapeDtypeStruct((N, D), jnp.float32),
    )


_tc_relu = _make_tc_combine(True)
_tc_plain = _make_tc_combine(False)


def kernel(x, edge_index, W_l1, W_r1, b1, W_l2, W_r2, b2, W_l3, W_r3, b3):
    src = edge_index[0]
    dst = edge_index[1]
    srcp = jnp.concatenate([src, jnp.zeros((PADE - E,), jnp.int32)])
    dstp = jnp.concatenate([dst, jnp.full((PADE - E,), N, jnp.int32)])
    deg = _sc_deg(x, srcp, dstp)
    d0, d1 = deg[0][:, :16], deg[1][:, :16]
    part1 = _sc_agg(x, srcp, dstp)
    h1 = _tc_relu(part1[0], part1[1], d0, d1, x,
                  W_l1.T, W_r1.T, b1.reshape(1, D))
    part2 = _sc_agg(h1, srcp, dstp)
    h2 = _tc_relu(part2[0], part2[1], d0, d1, h1,
                  W_l2.T, W_r2.T, b2.reshape(1, D))
    part3 = _sc_agg(h2, srcp, dstp)
    h3 = _tc_plain(part3[0], part3[1], d0, d1, h2,
                   W_l3.T, W_r3.T, b3.reshape(1, D))
    return h3
